# baseline (speedup 1.0000x reference)
import functools

import jax
import jax.numpy as jnp
from jax import lax
from jax.experimental import pallas as pl
from jax.experimental.pallas import tpu as pltpu
from jax.experimental.pallas import tpu_sc as plsc

N_HEADS = 12
N_KV_HEADS = 4
N_EMBD = 768
N_EXPERTS = 8
EPS = 1e-06
HD = N_EMBD // N_HEADS
N_REP = N_HEADS // N_KV_HEADS
KV_D = N_KV_HEADS * HD
S = 2048
ST = 256
NT = S // ST


def _rms(x, w):
    return x * jax.lax.rsqrt(jnp.mean(x * x, axis=-1, keepdims=True) + EPS) * w


def _k1_body(x_ref, nw_ref, wq_ref, wk_ref, wv_ref, q_ref, k_ref, v_ref):
    h = _rms(x_ref[...], nw_ref[0])
    q_ref[...] = jnp.dot(h, wq_ref[...], preferred_element_type=jnp.float32)
    k_ref[...] = jnp.dot(h, wk_ref[...], preferred_element_type=jnp.float32)
    v_ref[...] = jnp.dot(h, wv_ref[...], preferred_element_type=jnp.float32)


KT = 256


def _k2_body(q_ref, k_ref, v_ref, o_ref, acc_sc, m_sc, l_sc):
    qt = pl.program_id(0)
    kt = pl.program_id(1)

    @pl.when(kt == 0)
    def _():
        m_sc[...] = jnp.full((ST, 128), -1e30, jnp.float32)
        l_sc[...] = jnp.zeros((ST, 128), jnp.float32)

    @pl.when(kt <= qt)
    def _():
        rows = qt * ST + jax.lax.broadcasted_iota(jnp.int32, (ST, KT), 0)
        cols = kt * KT + jax.lax.broadcasted_iota(jnp.int32, (ST, KT), 1)
        causal = cols <= rows
        for h in range(N_HEADS):
            hs = slice(h * HD, (h + 1) * HD)
            kv = (h // N_REP) * HD
            q = q_ref[:, hs]
            kk = k_ref[pl.ds(kt * KT, KT), kv:kv + HD]
            s = jax.lax.dot_general(q, kk, (((1,), (1,)), ((), ())),
                                    preferred_element_type=jnp.float32)
            s = jnp.where(causal, s * (1.0 / HD ** 0.5), -1e30)
            m_old = m_sc[:, h:h + 1]
            mn = jnp.maximum(m_old, jnp.max(s, axis=-1, keepdims=True))
            p = jnp.exp(s - mn)
            r = jnp.exp(m_old - mn)
            l_sc[:, h:h + 1] = (l_sc[:, h:h + 1] * r
                                + jnp.sum(p, axis=-1, keepdims=True))
            base = jnp.where(kt == 0, jnp.zeros_like(q), acc_sc[:, hs])
            acc_sc[:, hs] = base * r + jnp.dot(
                p, v_ref[pl.ds(kt * KT, KT), kv:kv + HD],
                preferred_element_type=jnp.float32)
            m_sc[:, h:h + 1] = mn

    @pl.when(kt == NT - 1)
    def _():
        for h in range(N_HEADS):
            hs = slice(h * HD, (h + 1) * HD)
            o_ref[:, hs] = acc_sc[:, hs] / l_sc[:, h:h + 1]


def _k3_body(ao_ref, wo_ref, x_ref, fw_ref, gw_ref, h2_ref, self_ref):
    h2 = jnp.dot(ao_ref[...], wo_ref[...],
                 preferred_element_type=jnp.float32) + x_ref[...]
    h2_ref[...] = h2
    xn = _rms(h2, fw_ref[0])
    logits = jnp.dot(xn, gw_ref[...], preferred_element_type=jnp.float32)
    j = jax.lax.broadcasted_iota(jnp.int32, logits.shape, 1)
    logits = jnp.where(j < N_EXPERTS, logits, -1e30)
    mx = jnp.max(logits, axis=-1, keepdims=True)
    self_ref[0, 0] = jnp.min(jnp.where(logits == mx, j, 2 ** 30),
                             axis=-1).astype(jnp.int32)


NC, NS, L = 2, 16, 16
NW = NC * NS
TPW = S // NW
TT = 128
NTT = S // TT


def _sc_hist_body(sel_hbm, hist_hbm, sel_v, hist_mine):
    w = lax.axis_index("s") * NC + lax.axis_index("c")
    base = w * TPW
    iota = lax.iota(jnp.int32, L)
    pltpu.sync_copy(sel_hbm.at[pl.ds(base, TPW)], sel_v)
    one = jnp.full((L,), 1, jnp.int32)
    onehot = [one - jnp.minimum(jnp.abs(iota - e), one)
              for e in range(N_EXPERTS)]
    hist = jnp.zeros((L,), jnp.int32)
    for c in range(TPW // L):
        s = sel_v[pl.ds(c * L, L)]
        for e in range(N_EXPERTS):
            ind = one - jnp.minimum(jnp.abs(s - e), one)
            hist = hist + onehot[e] * jnp.sum(ind)
    hist_mine[...] = hist
    pltpu.sync_copy(hist_mine, hist_hbm.at[w])


def _sc_route_body(sel_hbm, h2_hbm, hist_hbm, h2s_hbm, pos_hbm, off_hbm,
                   sel_v, pos_v, tmp_v, hist_all, rows_v, sem):
    w = lax.axis_index("s") * NC + lax.axis_index("c")
    base = w * TPW
    iota = lax.iota(jnp.int32, L)

    pltpu.sync_copy(sel_hbm.at[pl.ds(base, TPW)], sel_v)
    pltpu.sync_copy(hist_hbm, hist_all)

    one = jnp.full((L,), 1, jnp.int32)
    onehot = [one - jnp.minimum(jnp.abs(iota - e), one)
              for e in range(N_EXPERTS)]

    totals = jnp.zeros((L,), jnp.int32)
    prefw = jnp.zeros((L,), jnp.int32)
    for wp in range(NW):
        row = hist_all[wp]
        totals = totals + row
        lt = jnp.minimum(jnp.maximum(w - wp, 0), 1)
        prefw = prefw + row * lt
    excl = plsc.cumsum(totals) - totals
    offw = excl + prefw

    @pl.when(w == 0)
    def _():
        tmp_v[...] = excl
        pltpu.sync_copy(tmp_v, off_hbm)

    running = offw
    for c in range(TPW // L):
        s = sel_v[pl.ds(c * L, L)]
        pos_c = jnp.zeros((L,), jnp.int32)
        for e in range(N_EXPERTS):
            ind = one - jnp.minimum(jnp.abs(s - e), one)
            mc = plsc.cumsum(ind)
            start = jnp.sum(running * onehot[e])
            pos_c = pos_c + ind * (start + mc - 1)
            running = running + onehot[e] * jnp.sum(ind)
        pos_v[pl.ds(c * L, L)] = pos_c
    pltpu.sync_copy(pos_v, pos_hbm.at[pl.ds(base, TPW)])

    pltpu.sync_copy(h2_hbm.at[pl.ds(base, TPW)], rows_v)
    pltpu.async_copy(rows_v, h2s_hbm.at[pos_v], sem).wait()


def _sc_unsort_body(os_hbm, pos_hbm, out_hbm, idx_v, rows_v, sem):
    w = lax.axis_index("s") * NC + lax.axis_index("c")
    base = w * TPW
    pltpu.sync_copy(pos_hbm.at[pl.ds(base, TPW)], idx_v)
    pltpu.async_copy(os_hbm.at[idx_v], rows_v, sem).wait()
    pltpu.sync_copy(rows_v, out_hbm.at[pl.ds(base, TPW)])


def _sc_route(sel, h2):
    mesh = plsc.VectorSubcoreMesh(core_axis_name="c", subcore_axis_name="s")
    hist = pl.kernel(
        _sc_hist_body,
        out_type=jax.ShapeDtypeStruct((NW, L), jnp.int32),
        mesh=mesh,
        scratch_types=[
            pltpu.VMEM((TPW,), jnp.int32),
            pltpu.VMEM((L,), jnp.int32),
        ],
        compiler_params=pltpu.CompilerParams(needs_layout_passes=False),
    )(sel)
    f = pl.kernel(
        _sc_route_body,
        out_type=[
            jax.ShapeDtypeStruct((S, N_EMBD), jnp.float32),
            jax.ShapeDtypeStruct((S,), jnp.int32),
            jax.ShapeDtypeStruct((L,), jnp.int32),
        ],
        mesh=mesh,
        scratch_types=[
            pltpu.VMEM((TPW,), jnp.int32),
            pltpu.VMEM((TPW,), jnp.int32),
            pltpu.VMEM((L,), jnp.int32),
            pltpu.VMEM((NW, L), jnp.int32),
            pltpu.VMEM((TPW, N_EMBD), jnp.float32),
            pltpu.SemaphoreType.DMA,
        ],
        compiler_params=pltpu.CompilerParams(needs_layout_passes=False),
    )
    return f(sel, h2, hist)


def _sc_unsort(out_sorted, pos):
    mesh = plsc.VectorSubcoreMesh(core_axis_name="c", subcore_axis_name="s")
    f = pl.kernel(
        _sc_unsort_body,
        out_type=jax.ShapeDtypeStruct((S, N_EMBD), jnp.float32),
        mesh=mesh,
        scratch_types=[
            pltpu.VMEM((TPW,), jnp.int32),
            pltpu.VMEM((TPW, N_EMBD), jnp.float32),
            pltpu.SemaphoreType.DMA,
        ],
        compiler_params=pltpu.CompilerParams(needs_layout_passes=False),
    )
    return f(out_sorted, pos)


def _k4_body(off_ref, h2s_ref, w_ref, fw_ref, out_ref):
    e = pl.program_id(0)
    t = pl.program_id(1)
    t0 = t * TT
    start = off_ref[e]
    end = off_ref[e + 1]

    @pl.when(jnp.logical_and(start < t0 + TT, end > t0))
    def _():
        rows = h2s_ref[pl.ds(t0, TT), :]
        xn = _rms(rows, fw_ref[0])
        y = jnp.dot(xn, w_ref[0], preferred_element_type=jnp.float32)
        rid = t0 + jax.lax.broadcasted_iota(jnp.int32, (TT, 1), 0)
        mask = jnp.logical_and(rid >= start, rid < end)
        prev = out_ref[pl.ds(t0, TT), :]
        out_ref[pl.ds(t0, TT), :] = jnp.where(mask, rows + y, prev)


def kernel(x, attn_norm_w, wq, wk, wv, wo, ffn_norm_w, gate_w, expert_w):
    B, S_, C = x.shape
    x2 = x.reshape(S_, C)
    nw = attn_norm_w.reshape(1, C)
    fw = ffn_norm_w.reshape(1, C)
    gwp = jnp.pad(gate_w, ((0, 0), (0, 128 - N_EXPERTS)))

    q, k, v = pl.pallas_call(
        _k1_body,
        grid=(NT,),
        in_specs=[
            pl.BlockSpec((ST, C), lambda t: (t, 0)),
            pl.BlockSpec((1, C), lambda t: (0, 0)),
            pl.BlockSpec((C, C), lambda t: (0, 0)),
            pl.BlockSpec((C, KV_D), lambda t: (0, 0)),
            pl.BlockSpec((C, KV_D), lambda t: (0, 0)),
        ],
        out_specs=[
            pl.BlockSpec((ST, C), lambda t: (t, 0)),
            pl.BlockSpec((ST, KV_D), lambda t: (t, 0)),
            pl.BlockSpec((ST, KV_D), lambda t: (t, 0)),
        ],
        out_shape=[
            jax.ShapeDtypeStruct((S_, C), jnp.float32),
            jax.ShapeDtypeStruct((S_, KV_D), jnp.float32),
            jax.ShapeDtypeStruct((S_, KV_D), jnp.float32),
        ],
    )(x2, nw, wq, wk, wv)

    ao2 = pl.pallas_call(
        _k2_body,
        grid=(NT, NT),
        in_specs=[
            pl.BlockSpec((ST, C), lambda t, j: (t, 0)),
            pl.BlockSpec((S_, KV_D), lambda t, j: (0, 0)),
            pl.BlockSpec((S_, KV_D), lambda t, j: (0, 0)),
        ],
        out_specs=pl.BlockSpec((ST, C), lambda t, j: (t, 0)),
        out_shape=jax.ShapeDtypeStruct((S_, C), jnp.float32),
        scratch_shapes=[
            pltpu.VMEM((ST, C), jnp.float32),
            pltpu.VMEM((ST, 128), jnp.float32),
            pltpu.VMEM((ST, 128), jnp.float32),
        ],
    )(q, k, v)

    h2, sel3 = pl.pallas_call(
        _k3_body,
        grid=(NT,),
        in_specs=[
            pl.BlockSpec((ST, C), lambda t: (t, 0)),
            pl.BlockSpec((C, C), lambda t: (0, 0)),
            pl.BlockSpec((ST, C), lambda t: (t, 0)),
            pl.BlockSpec((1, C), lambda t: (0, 0)),
            pl.BlockSpec((C, 128), lambda t: (0, 0)),
        ],
        out_specs=[
            pl.BlockSpec((ST, C), lambda t: (t, 0)),
            pl.BlockSpec((1, 1, ST), lambda t: (t, 0, 0)),
        ],
        out_shape=[
            jax.ShapeDtypeStruct((S_, C), jnp.float32),
            jax.ShapeDtypeStruct((NT, 1, ST), jnp.int32),
        ],
    )(ao2, wo, x2, fw, gwp)

    sel = sel3.reshape(S_)
    h2s, pos, off = _sc_route(sel, h2)

    out_sorted = pl.pallas_call(
        _k4_body,
        grid_spec=pltpu.PrefetchScalarGridSpec(
            num_scalar_prefetch=1,
            grid=(N_EXPERTS, NTT),
            in_specs=[
                pl.BlockSpec((S_, C), lambda e, t, off_r: (0, 0)),
                pl.BlockSpec((1, C, C), lambda e, t, off_r: (e, 0, 0)),
                pl.BlockSpec((1, C), lambda e, t, off_r: (0, 0)),
            ],
            out_specs=pl.BlockSpec((S_, C), lambda e, t, off_r: (0, 0)),
        ),
        out_shape=jax.ShapeDtypeStruct((S_, C), jnp.float32),
    )(off, h2s, expert_w, fw)

    out = _sc_unsort(out_sorted, pos)

    return out.reshape(B, S_, C)


# speedup vs baseline: 2.1919x; 2.1919x over previous
import functools

import jax
import jax.numpy as jnp
from jax import lax
from jax.experimental import pallas as pl
from jax.experimental.pallas import tpu as pltpu
from jax.experimental.pallas import tpu_sc as plsc

N_HEADS = 12
N_KV_HEADS = 4
N_EMBD = 768
N_EXPERTS = 8
EPS = 1e-06
HD = N_EMBD // N_HEADS
N_REP = N_HEADS // N_KV_HEADS
KV_D = N_KV_HEADS * HD
S = 2048
ST = 256
NT = S // ST


def _rms(x, w):
    return x * jax.lax.rsqrt(jnp.mean(x * x, axis=-1, keepdims=True) + EPS) * w


def _k1_body(x_ref, nw_ref, wq_ref, wk_ref, wv_ref, q_ref, k_ref, v_ref):
    h = _rms(x_ref[...], nw_ref[0])
    q_ref[...] = jnp.dot(h, wq_ref[...],
                         preferred_element_type=jnp.float32) * (1.0 / HD ** 0.5)
    k_ref[...] = jnp.dot(h, wk_ref[...], preferred_element_type=jnp.float32)
    v_ref[...] = jnp.dot(h, wv_ref[...], preferred_element_type=jnp.float32)


def _k2s_body(q_ref, k_ref, v_ref, o_ref, *, t):
    KL = (t + 1) * ST
    rows = t * ST + jax.lax.broadcasted_iota(jnp.int32, (ST, KL), 0)
    cols = jax.lax.broadcasted_iota(jnp.int32, (ST, KL), 1)
    causal = cols <= rows
    for h in range(N_HEADS):
        q = q_ref[:, h * HD:(h + 1) * HD]
        kv = (h // N_REP) * HD
        k = k_ref[:, kv:kv + HD]
        s = jax.lax.dot_general(q, k, (((1,), (1,)), ((), ())),
                                preferred_element_type=jnp.float32)
        s = jnp.where(causal, s, -1e30)
        m = jnp.max(s, axis=-1, keepdims=True)
        p = jnp.exp(s - m)
        rl = 1.0 / jnp.sum(p, axis=-1, keepdims=True)
        o_ref[:, h * HD:(h + 1) * HD] = jnp.dot(
            p, v_ref[:, kv:kv + HD], preferred_element_type=jnp.float32) * rl


def _k3_body(ao_ref, wo_ref, x_ref, fw_ref, gw_ref, h2_ref, self_ref):
    h2 = jnp.dot(ao_ref[...], wo_ref[...],
                 preferred_element_type=jnp.float32) + x_ref[...]
    h2_ref[...] = h2
    xn = _rms(h2, fw_ref[0])
    logits = jnp.dot(xn, gw_ref[...], preferred_element_type=jnp.float32)
    j = jax.lax.broadcasted_iota(jnp.int32, logits.shape, 1)
    logits = jnp.where(j < N_EXPERTS, logits, -1e30)
    mx = jnp.max(logits, axis=-1, keepdims=True)
    self_ref[0, 0] = jnp.min(jnp.where(logits == mx, j, 2 ** 30),
                             axis=-1).astype(jnp.int32)


NC, NS, L = 2, 16, 16
NW = NC * NS
TPW = S // NW
TT = 128
NTT = S // TT


def _sc_hist_body(sel_hbm, hist_hbm, sel_v, hist_mine):
    w = lax.axis_index("s") * NC + lax.axis_index("c")
    base = w * TPW
    iota = lax.iota(jnp.int32, L)
    pltpu.sync_copy(sel_hbm.at[pl.ds(base, TPW)], sel_v)
    one = jnp.full((L,), 1, jnp.int32)
    onehot = [one - jnp.minimum(jnp.abs(iota - e), one)
              for e in range(N_EXPERTS)]
    hist = jnp.zeros((L,), jnp.int32)
    for c in range(TPW // L):
        s = sel_v[pl.ds(c * L, L)]
        for e in range(N_EXPERTS):
            ind = one - jnp.minimum(jnp.abs(s - e), one)
            hist = hist + onehot[e] * jnp.sum(ind)
    hist_mine[...] = hist
    pltpu.sync_copy(hist_mine, hist_hbm.at[w])


def _sc_route_body(sel_hbm, h2_hbm, hist_hbm, h2s_hbm, pos_hbm, off_hbm,
                   sel_v, pos_v, tmp_v, hist_all, rows_v, sem):
    w = lax.axis_index("s") * NC + lax.axis_index("c")
    base = w * TPW
    iota = lax.iota(jnp.int32, L)

    pltpu.sync_copy(sel_hbm.at[pl.ds(base, TPW)], sel_v)
    pltpu.sync_copy(hist_hbm, hist_all)

    one = jnp.full((L,), 1, jnp.int32)
    onehot = [one - jnp.minimum(jnp.abs(iota - e), one)
              for e in range(N_EXPERTS)]

    totals = jnp.zeros((L,), jnp.int32)
    prefw = jnp.zeros((L,), jnp.int32)
    for wp in range(NW):
        row = hist_all[wp]
        totals = totals + row
        lt = jnp.minimum(jnp.maximum(w - wp, 0), 1)
        prefw = prefw + row * lt
    excl = plsc.cumsum(totals) - totals
    offw = excl + prefw

    @pl.when(w == 0)
    def _():
        tmp_v[...] = excl
        pltpu.sync_copy(tmp_v, off_hbm)

    running = offw
    for c in range(TPW // L):
        s = sel_v[pl.ds(c * L, L)]
        pos_c = jnp.zeros((L,), jnp.int32)
        for e in range(N_EXPERTS):
            ind = one - jnp.minimum(jnp.abs(s - e), one)
            mc = plsc.cumsum(ind)
            start = jnp.sum(running * onehot[e])
            pos_c = pos_c + ind * (start + mc - 1)
            running = running + onehot[e] * jnp.sum(ind)
        pos_v[pl.ds(c * L, L)] = pos_c
    pltpu.sync_copy(pos_v, pos_hbm.at[pl.ds(base, TPW)])

    pltpu.sync_copy(h2_hbm.at[pl.ds(base, TPW)], rows_v)
    pltpu.async_copy(rows_v, h2s_hbm.at[pos_v], sem).wait()


def _sc_unsort_body(os_hbm, pos_hbm, out_hbm, idx_v, rows_v, sem):
    w = lax.axis_index("s") * NC + lax.axis_index("c")
    base = w * TPW
    pltpu.sync_copy(pos_hbm.at[pl.ds(base, TPW)], idx_v)
    pltpu.async_copy(os_hbm.at[idx_v], rows_v, sem).wait()
    pltpu.sync_copy(rows_v, out_hbm.at[pl.ds(base, TPW)])


def _sc_route(sel, h2):
    mesh = plsc.VectorSubcoreMesh(core_axis_name="c", subcore_axis_name="s")
    hist = pl.kernel(
        _sc_hist_body,
        out_type=jax.ShapeDtypeStruct((NW, L), jnp.int32),
        mesh=mesh,
        scratch_types=[
            pltpu.VMEM((TPW,), jnp.int32),
            pltpu.VMEM((L,), jnp.int32),
        ],
        compiler_params=pltpu.CompilerParams(needs_layout_passes=False),
    )(sel)
    f = pl.kernel(
        _sc_route_body,
        out_type=[
            jax.ShapeDtypeStruct((S, N_EMBD), jnp.float32),
            jax.ShapeDtypeStruct((S,), jnp.int32),
            jax.ShapeDtypeStruct((L,), jnp.int32),
        ],
        mesh=mesh,
        scratch_types=[
            pltpu.VMEM((TPW,), jnp.int32),
            pltpu.VMEM((TPW,), jnp.int32),
            pltpu.VMEM((L,), jnp.int32),
            pltpu.VMEM((NW, L), jnp.int32),
            pltpu.VMEM((TPW, N_EMBD), jnp.float32),
            pltpu.SemaphoreType.DMA,
        ],
        compiler_params=pltpu.CompilerParams(needs_layout_passes=False),
    )
    return f(sel, h2, hist)


def _sc_unsort(out_sorted, pos):
    mesh = plsc.VectorSubcoreMesh(core_axis_name="c", subcore_axis_name="s")
    f = pl.kernel(
        _sc_unsort_body,
        out_type=jax.ShapeDtypeStruct((S, N_EMBD), jnp.float32),
        mesh=mesh,
        scratch_types=[
            pltpu.VMEM((TPW,), jnp.int32),
            pltpu.VMEM((TPW, N_EMBD), jnp.float32),
            pltpu.SemaphoreType.DMA,
        ],
        compiler_params=pltpu.CompilerParams(needs_layout_passes=False),
    )
    return f(out_sorted, pos)


def _k4_body(off_ref, h2s_ref, w_ref, fw_ref, out_ref):
    e = pl.program_id(0)
    t = pl.program_id(1)
    t0 = t * TT
    start = off_ref[e]
    end = off_ref[e + 1]

    @pl.when(jnp.logical_and(start < t0 + TT, end > t0))
    def _():
        rows = h2s_ref[pl.ds(t0, TT), :]
        xn = _rms(rows, fw_ref[0])
        y = jnp.dot(xn, w_ref[0], preferred_element_type=jnp.float32)
        rid = t0 + jax.lax.broadcasted_iota(jnp.int32, (TT, 1), 0)
        mask = jnp.logical_and(rid >= start, rid < end)
        prev = out_ref[pl.ds(t0, TT), :]
        out_ref[pl.ds(t0, TT), :] = jnp.where(mask, rows + y, prev)


def kernel(x, attn_norm_w, wq, wk, wv, wo, ffn_norm_w, gate_w, expert_w):
    B, S_, C = x.shape
    x2 = x.reshape(S_, C)
    nw = attn_norm_w.reshape(1, C)
    fw = ffn_norm_w.reshape(1, C)
    gwp = jnp.pad(gate_w, ((0, 0), (0, 128 - N_EXPERTS)))

    q, k, v = pl.pallas_call(
        _k1_body,
        grid=(NT,),
        in_specs=[
            pl.BlockSpec((ST, C), lambda t: (t, 0)),
            pl.BlockSpec((1, C), lambda t: (0, 0)),
            pl.BlockSpec((C, C), lambda t: (0, 0)),
            pl.BlockSpec((C, KV_D), lambda t: (0, 0)),
            pl.BlockSpec((C, KV_D), lambda t: (0, 0)),
        ],
        out_specs=[
            pl.BlockSpec((ST, C), lambda t: (t, 0)),
            pl.BlockSpec((ST, KV_D), lambda t: (t, 0)),
            pl.BlockSpec((ST, KV_D), lambda t: (t, 0)),
        ],
        out_shape=[
            jax.ShapeDtypeStruct((S_, C), jnp.float32),
            jax.ShapeDtypeStruct((S_, KV_D), jnp.float32),
            jax.ShapeDtypeStruct((S_, KV_D), jnp.float32),
        ],
    )(x2, nw, wq, wk, wv)

    ao_tiles = []
    for t in range(NT):
        KL = (t + 1) * ST
        ao_t = pl.pallas_call(
            functools.partial(_k2s_body, t=t),
            grid=(1,),
            in_specs=[
                pl.BlockSpec((ST, C), lambda i, t=t: (t, 0)),
                pl.BlockSpec((KL, KV_D), lambda i: (0, 0)),
                pl.BlockSpec((KL, KV_D), lambda i: (0, 0)),
            ],
            out_specs=pl.BlockSpec((ST, C), lambda i: (0, 0)),
            out_shape=jax.ShapeDtypeStruct((ST, C), jnp.float32),
        )(q, k, v)
        ao_tiles.append(ao_t)
    ao2 = jnp.concatenate(ao_tiles, axis=0)

    h2, sel3 = pl.pallas_call(
        _k3_body,
        grid=(NT,),
        in_specs=[
            pl.BlockSpec((ST, C), lambda t: (t, 0)),
            pl.BlockSpec((C, C), lambda t: (0, 0)),
            pl.BlockSpec((ST, C), lambda t: (t, 0)),
            pl.BlockSpec((1, C), lambda t: (0, 0)),
            pl.BlockSpec((C, 128), lambda t: (0, 0)),
        ],
        out_specs=[
            pl.BlockSpec((ST, C), lambda t: (t, 0)),
            pl.BlockSpec((1, 1, ST), lambda t: (t, 0, 0)),
        ],
        out_shape=[
            jax.ShapeDtypeStruct((S_, C), jnp.float32),
            jax.ShapeDtypeStruct((NT, 1, ST), jnp.int32),
        ],
    )(ao2, wo, x2, fw, gwp)

    sel = sel3.reshape(S_)
    h2s, pos, off = _sc_route(sel, h2)

    out_sorted = pl.pallas_call(
        _k4_body,
        grid_spec=pltpu.PrefetchScalarGridSpec(
            num_scalar_prefetch=1,
            grid=(N_EXPERTS, NTT),
            in_specs=[
                pl.BlockSpec((S_, C), lambda e, t, off_r: (0, 0)),
                pl.BlockSpec((1, C, C), lambda e, t, off_r: (e, 0, 0)),
                pl.BlockSpec((1, C), lambda e, t, off_r: (0, 0)),
            ],
            out_specs=pl.BlockSpec((S_, C), lambda e, t, off_r: (0, 0)),
        ),
        out_shape=jax.ShapeDtypeStruct((S_, C), jnp.float32),
    )(off, h2s, expert_w, fw)

    out = _sc_unsort(out_sorted, pos)

    return out.reshape(B, S_, C)
